# baseline (speedup 1.0000x reference)
import jax
import jax.numpy as jnp
from jax.experimental import pallas as pl
from jax.experimental.pallas import tpu as pltpu

N = 4096
BLK = 512
NBLK = N // BLK
XBLK = 256
NXBLK = N // XBLK
GRID = NXBLK + NBLK


def _gcn_kernel(x_ref, adj_ref, w1_ref, b1_ref, w2bar_ref, b2bar_ref,
                out_ref, xw_scr, adjc_scr, vb_scr, oacc_scr):
    i = pl.program_id(0)

    @pl.when(i < NXBLK)
    def _():
        xw_scr[pl.ds(i * XBLK, XBLK), :] = jnp.dot(
            x_ref[...], w1_ref[...],
            preferred_element_type=jnp.float32).astype(jnp.bfloat16)
        adjc_scr[pl.ds(i * XBLK, XBLK), :] = jnp.zeros((XBLK, N),
                                                       jnp.bfloat16)

    @pl.when(i == 0)
    def _():
        vb_scr[...] = jnp.zeros((N, 1), jnp.bfloat16)
        oacc_scr[...] = jnp.zeros((N, 1), jnp.float32)

    @pl.when(i >= NXBLK)
    def _():
        j = i - NXBLK
        rs = pl.ds(j * BLK, BLK)
        adjc_scr[rs, :] = adj_ref[...].astype(jnp.bfloat16)
        h = jnp.dot(adjc_scr[rs, :], xw_scr[...],
                    preferred_element_type=jnp.float32)
        h = jnp.maximum(h + b1_ref[...], 0.0)
        oacc_scr[rs, :] += jnp.dot(adjc_scr[rs, :], vb_scr[...],
                                   preferred_element_type=jnp.float32)
        vcol = jax.lax.dot_general(
            h, w2bar_ref[...], (((1,), (1,)), ((), ())),
            preferred_element_type=jnp.float32)
        vb = vcol.astype(jnp.bfloat16)
        vb_scr[rs, :] = vb
        oacc_scr[...] += jnp.dot(adjc_scr[:, pl.ds(j * BLK, BLK)], vb,
                                 preferred_element_type=jnp.float32)

    @pl.when(i == GRID - 1)
    def _():
        out_ref[...] = oacc_scr[...] + b2bar_ref[0, 0]


def kernel(x, adj, W1, b1, W2, b2):
    feat = x.shape[1]
    hidden = W1.shape[1]
    w2bar = jnp.mean(W2, axis=1).reshape(1, hidden)
    b2bar = jnp.mean(b2).reshape(1, 1)
    b1r = b1.reshape(1, hidden)

    out = pl.pallas_call(
        _gcn_kernel,
        grid=(GRID,),
        in_specs=[
            pl.BlockSpec((XBLK, feat),
                         lambda i: (jnp.minimum(i, NXBLK - 1), 0)),
            pl.BlockSpec((BLK, N),
                         lambda i: (jnp.clip(i - NXBLK, 0, NBLK - 1), 0)),
            pl.BlockSpec((feat, hidden), lambda i: (0, 0)),
            pl.BlockSpec((1, hidden), lambda i: (0, 0)),
            pl.BlockSpec((1, hidden), lambda i: (0, 0)),
            pl.BlockSpec((1, 1), lambda i: (0, 0)),
        ],
        out_specs=pl.BlockSpec((N, 1), lambda i: (0, 0)),
        out_shape=jax.ShapeDtypeStruct((N, 1), jnp.float32),
        scratch_shapes=[
            pltpu.VMEM((N, hidden), jnp.bfloat16),
            pltpu.VMEM((N, N), jnp.bfloat16),
            pltpu.VMEM((N, 1), jnp.bfloat16),
            pltpu.VMEM((N, 1), jnp.float32),
        ],
        compiler_params=pltpu.CompilerParams(
            dimension_semantics=("arbitrary",),
            vmem_limit_bytes=100 * 1024 * 1024,
        ),
    )(x, adj, W1, b1r, w2bar, b2bar)

    return out.reshape(N)


# speedup vs baseline: 1.1056x; 1.1056x over previous
import jax
import jax.numpy as jnp
from jax.experimental import pallas as pl
from jax.experimental.pallas import tpu as pltpu

N = 4096
BLK = 256
NBLK = N // BLK
XBLK = 512
NXBLK = N // XBLK
TROWS = 1024
NT = N // TROWS
TC = 512
GRID = NXBLK + NBLK + NT


def _gcn_kernel(x_ref, adj_ref, w1_ref, b1_ref, w2bar_ref, b2bar_ref,
                out_ref, xw_scr, adjc_scr, vb_scr):
    i = pl.program_id(0)

    @pl.when(i < NXBLK)
    def _():
        xw_scr[pl.ds(i * XBLK, XBLK), :] = jnp.dot(
            x_ref[...], w1_ref[...],
            preferred_element_type=jnp.float32).astype(jnp.bfloat16)

    @pl.when(jnp.logical_and(i >= NXBLK, i < NXBLK + NBLK))
    def _():
        j = i - NXBLK
        rs = pl.ds(j * BLK, BLK)
        adjc_scr[rs, :] = adj_ref[...].astype(jnp.bfloat16)
        h = jnp.dot(adjc_scr[rs, :], xw_scr[...],
                    preferred_element_type=jnp.float32)
        h = jnp.maximum(h + b1_ref[...], 0.0)
        vcol = jax.lax.dot_general(
            h, w2bar_ref[...], (((1,), (1,)), ((), ())),
            preferred_element_type=jnp.float32)
        vb_scr[rs, :] = vcol.astype(jnp.bfloat16)

    @pl.when(i >= NXBLK + NBLK)
    def _():
        t = i - (NXBLK + NBLK)
        rs = pl.ds(t * TROWS, TROWS)
        acc = jnp.full((TROWS, 1), b2bar_ref[0, 0], jnp.float32)
        for c in range(N // TC):
            acc += jnp.dot(adjc_scr[rs, c * TC:(c + 1) * TC],
                           vb_scr[c * TC:(c + 1) * TC, :],
                           preferred_element_type=jnp.float32)
        out_ref[rs, :] = acc


def kernel(x, adj, W1, b1, W2, b2):
    feat = x.shape[1]
    hidden = W1.shape[1]
    w2bar = jnp.mean(W2, axis=1).reshape(1, hidden)
    b2bar = jnp.mean(b2).reshape(1, 1)
    b1r = b1.reshape(1, hidden)

    out = pl.pallas_call(
        _gcn_kernel,
        grid=(GRID,),
        in_specs=[
            pl.BlockSpec((XBLK, feat),
                         lambda i: (jnp.minimum(i, NXBLK - 1), 0)),
            pl.BlockSpec((BLK, N),
                         lambda i: (jnp.clip(i - NXBLK, 0, NBLK - 1), 0)),
            pl.BlockSpec((feat, hidden), lambda i: (0, 0)),
            pl.BlockSpec((1, hidden), lambda i: (0, 0)),
            pl.BlockSpec((1, hidden), lambda i: (0, 0)),
            pl.BlockSpec((1, 1), lambda i: (0, 0)),
        ],
        out_specs=pl.BlockSpec((N, 1), lambda i: (0, 0)),
        out_shape=jax.ShapeDtypeStruct((N, 1), jnp.float32),
        scratch_shapes=[
            pltpu.VMEM((N, hidden), jnp.bfloat16),
            pltpu.VMEM((N, N), jnp.bfloat16),
            pltpu.VMEM((N, 1), jnp.bfloat16),
        ],
        compiler_params=pltpu.CompilerParams(
            dimension_semantics=("arbitrary",),
            vmem_limit_bytes=100 * 1024 * 1024,
        ),
    )(x, adj, W1, b1r, w2bar, b2bar)

    return out.reshape(N)


# speedup vs baseline: 1.1840x; 1.0709x over previous
import jax
import jax.numpy as jnp
from jax.experimental import pallas as pl
from jax.experimental.pallas import tpu as pltpu

N = 4096
BLK = 512
NBLK = N // BLK
XBLK = 512
NXBLK = N // XBLK
TROWS = 1024
NT = N // TROWS
TC = 512
GRID = NXBLK + NBLK + NT


def _gcn_kernel(x_ref, adj_ref, w1_ref, b1_ref, w2bar_ref, b2bar_ref,
                out_ref, xw_scr, adjc_scr, vb_scr):
    i = pl.program_id(0)

    @pl.when(i < NXBLK)
    def _():
        xw_scr[pl.ds(i * XBLK, XBLK), :] = jnp.dot(
            x_ref[...], w1_ref[...],
            preferred_element_type=jnp.float32).astype(jnp.bfloat16)

    @pl.when(jnp.logical_and(i >= NXBLK, i < NXBLK + NBLK))
    def _():
        j = i - NXBLK
        rs = pl.ds(j * BLK, BLK)
        adjc_scr[rs, :] = adj_ref[...].astype(jnp.bfloat16)
        h = jnp.dot(adjc_scr[rs, :], xw_scr[...],
                    preferred_element_type=jnp.float32)
        h = jnp.maximum(h + b1_ref[...], 0.0)
        vcol = jax.lax.dot_general(
            h, w2bar_ref[...], (((1,), (1,)), ((), ())),
            preferred_element_type=jnp.float32)
        vb_scr[rs, :] = vcol.astype(jnp.bfloat16)

    @pl.when(i >= NXBLK + NBLK)
    def _():
        t = i - (NXBLK + NBLK)
        rs = pl.ds(t * TROWS, TROWS)
        acc = jnp.full((TROWS, 1), b2bar_ref[0, 0], jnp.float32)
        for c in range(N // TC):
            acc += jnp.dot(adjc_scr[rs, c * TC:(c + 1) * TC],
                           vb_scr[c * TC:(c + 1) * TC, :],
                           preferred_element_type=jnp.float32)
        out_ref[rs, :] = acc


def kernel(x, adj, W1, b1, W2, b2):
    feat = x.shape[1]
    hidden = W1.shape[1]
    w2bar = jnp.mean(W2, axis=1).reshape(1, hidden)
    b2bar = jnp.mean(b2).reshape(1, 1)
    b1r = b1.reshape(1, hidden)

    out = pl.pallas_call(
        _gcn_kernel,
        grid=(GRID,),
        in_specs=[
            pl.BlockSpec((XBLK, feat),
                         lambda i: (jnp.minimum(i, NXBLK - 1), 0)),
            pl.BlockSpec((BLK, N),
                         lambda i: (jnp.clip(i - NXBLK, 0, NBLK - 1), 0)),
            pl.BlockSpec((feat, hidden), lambda i: (0, 0)),
            pl.BlockSpec((1, hidden), lambda i: (0, 0)),
            pl.BlockSpec((1, hidden), lambda i: (0, 0)),
            pl.BlockSpec((1, 1), lambda i: (0, 0)),
        ],
        out_specs=pl.BlockSpec((N, 1), lambda i: (0, 0)),
        out_shape=jax.ShapeDtypeStruct((N, 1), jnp.float32),
        scratch_shapes=[
            pltpu.VMEM((N, hidden), jnp.bfloat16),
            pltpu.VMEM((N, N), jnp.bfloat16),
            pltpu.VMEM((N, 1), jnp.bfloat16),
        ],
        compiler_params=pltpu.CompilerParams(
            dimension_semantics=("arbitrary",),
            vmem_limit_bytes=100 * 1024 * 1024,
        ),
    )(x, adj, W1, b1r, w2bar, b2bar)

    return out.reshape(N)
